# baseline (speedup 1.0000x reference)
import functools

import jax
import jax.numpy as jnp
from jax import lax
from jax.experimental import pallas as pl
from jax.experimental.pallas import tpu as pltpu
from jax.experimental.pallas import tpu_sc as plsc

N = 100
D = 131072
NSTRIPE = 4
W = D // NSTRIPE
S = 128
LOG2_S = 7
D2 = W // S
B2 = N * S
NW = 32
PER_W = B2 // NW
K = 16
STEPS = PER_W // K
NBUF = 4
L = 16

_mesh = plsc.VectorSubcoreMesh(core_axis_name="c", subcore_axis_name="s")


@functools.partial(
    pl.kernel,
    out_type=jax.ShapeDtypeStruct((B2, D2), jnp.float32),
    mesh=_mesh,
    compiler_params=pltpu.CompilerParams(needs_layout_passes=False),
    scratch_types=[
        pltpu.VMEM((128,), jnp.int32),
        pltpu.VMEM((PER_W,), jnp.int32),
        [pltpu.VMEM((K, D2), jnp.float32) for _ in range(NBUF)],
        [pltpu.SemaphoreType.DMA for _ in range(NBUF)],
        [pltpu.SemaphoreType.DMA for _ in range(NBUF)],
    ],
)
def _sc_gather(x2, ords, out, ordv, idxv, bufs, gsems, wsems):
    cid = lax.axis_index("c")
    sid = lax.axis_index("s")
    wid = sid * 2 + cid
    base = wid * PER_W

    pltpu.sync_copy(ords, ordv.at[pl.ds(0, N)])

    for j0 in range(0, PER_W, L):
        r = base + j0 + lax.iota(jnp.int32, L)
        i = lax.shift_right_logical(r, LOG2_S)
        c = lax.bitwise_and(r, S - 1)
        ov = plsc.load_gather(ordv, [i])
        idxv[pl.ds(j0, L)] = ov * S + c

    def start_gather(s):
        b = s % NBUF
        pltpu.async_copy(x2.at[idxv.at[pl.ds(s * K, K)]], bufs[b], gsems[b])

    scats = [None] * NBUF
    for s in range(min(NBUF - 1, STEPS)):
        start_gather(s)
    for s in range(STEPS):
        b = s % NBUF
        pltpu.make_async_copy(x2.at[idxv.at[pl.ds(s * K, K)]],
                              bufs[b], gsems[b]).wait()
        scats[b] = pltpu.async_copy(bufs[b], out.at[pl.ds(base + s * K, K)],
                                    wsems[b])
        n = s + NBUF - 1
        if n < STEPS:
            nb = n % NBUF
            if scats[nb] is not None:
                scats[nb].wait()
                scats[nb] = None
            start_gather(n)
    for b in range(NBUF):
        if scats[b] is not None:
            scats[b].wait()


def kernel(x, ordinals):
    outs = []
    for k in range(NSTRIPE):
        xk = lax.slice(x, (0, k * W), (N, (k + 1) * W)).reshape(B2, D2)
        outs.append(_sc_gather(xk, ordinals).reshape(N, W))
    return jnp.concatenate(outs, axis=1)


# speedup vs baseline: 3.7095x; 3.7095x over previous
import functools

import jax
import jax.numpy as jnp
from jax import lax
from jax.experimental import pallas as pl
from jax.experimental.pallas import tpu as pltpu
from jax.experimental.pallas import tpu_sc as plsc

N = 100
D = 131072
NW = 32
WSTRIPE = D // NW
CW = 2048
NCC = WSTRIPE // CW
L = 16
ROW_WINDOWS = ((0, 16), (16, 16), (32, 16), (48, 16), (64, 16), (80, 16),
               (96, 4))
NBUF = 3

_mesh = plsc.VectorSubcoreMesh(core_axis_name="c", subcore_axis_name="s")


@functools.partial(
    pl.kernel,
    out_type=jax.ShapeDtypeStruct((N, D), jnp.float32),
    mesh=_mesh,
    compiler_params=pltpu.CompilerParams(needs_layout_passes=False),
    scratch_types=[
        pltpu.VMEM((128,), jnp.int32),
        [pltpu.VMEM((L, CW), jnp.float32) for _ in range(NBUF)],
        [pltpu.SemaphoreType.DMA for _ in range(NBUF)],
        [pltpu.SemaphoreType.DMA for _ in range(NBUF)],
    ],
)
def _sc_gather(x, ords, out, ordv, bufs, gsems, wsems):
    cid = lax.axis_index("c")
    sid = lax.axis_index("s")
    wid = sid * 2 + cid
    c0 = wid * WSTRIPE

    ordv[pl.ds(96, L)] = jnp.zeros((L,), jnp.int32)
    pltpu.sync_copy(ords, ordv.at[pl.ds(0, N)])

    units = [(w, r, cc) for (w, r) in ROW_WINDOWS for cc in range(NCC)]
    iota = lax.iota(jnp.int32, L)

    def start_gather(s):
        w, _, cc = units[s]
        idx = plsc.load_gather(ordv, [w + iota])
        src = x.at[:, pl.ds(c0 + cc * CW, CW)]
        pltpu.async_copy(src.at[idx], bufs[s % NBUF], gsems[s % NBUF])

    scats = [None] * NBUF
    for s in range(min(NBUF - 1, len(units))):
        start_gather(s)
    for s in range(len(units)):
        b = s % NBUF
        w, r, cc = units[s]
        src = x.at[:, pl.ds(c0 + cc * CW, CW)]
        idx = plsc.load_gather(ordv, [w + iota])
        pltpu.make_async_copy(src.at[idx], bufs[b], gsems[b]).wait()
        scats[b] = pltpu.async_copy(
            bufs[b].at[pl.ds(0, r)],
            out.at[pl.ds(w, r), pl.ds(c0 + cc * CW, CW)], wsems[b])
        n = s + NBUF - 1
        if n < len(units):
            nb = n % NBUF
            if scats[nb] is not None:
                scats[nb].wait()
                scats[nb] = None
            start_gather(n)
    for b in range(NBUF):
        if scats[b] is not None:
            scats[b].wait()


def kernel(x, ordinals):
    return _sc_gather(x, ordinals)


# speedup vs baseline: 3.7546x; 1.0122x over previous
import functools

import jax
import jax.numpy as jnp
from jax import lax
from jax.experimental import pallas as pl
from jax.experimental.pallas import tpu as pltpu
from jax.experimental.pallas import tpu_sc as plsc

N = 100
D = 131072
NW = 32
WSTRIPE = D // NW
CW = 1024
NCC = WSTRIPE // CW
L = 16
ROW_WINDOWS = ((0, 16), (16, 16), (32, 16), (48, 16), (64, 16), (80, 16),
               (96, 4))
NBUF = 6

_mesh = plsc.VectorSubcoreMesh(core_axis_name="c", subcore_axis_name="s")


@functools.partial(
    pl.kernel,
    out_type=jax.ShapeDtypeStruct((N, D), jnp.float32),
    mesh=_mesh,
    compiler_params=pltpu.CompilerParams(needs_layout_passes=False),
    scratch_types=[
        pltpu.VMEM((128,), jnp.int32),
        [pltpu.VMEM((L, CW), jnp.float32) for _ in range(NBUF)],
        [pltpu.SemaphoreType.DMA for _ in range(NBUF)],
        [pltpu.SemaphoreType.DMA for _ in range(NBUF)],
    ],
)
def _sc_gather(x, ords, out, ordv, bufs, gsems, wsems):
    cid = lax.axis_index("c")
    sid = lax.axis_index("s")
    wid = sid * 2 + cid
    c0 = wid * WSTRIPE

    ordv[pl.ds(96, L)] = jnp.zeros((L,), jnp.int32)
    pltpu.sync_copy(ords, ordv.at[pl.ds(0, N)])

    units = [(w, r, cc) for (w, r) in ROW_WINDOWS for cc in range(NCC)]
    iota = lax.iota(jnp.int32, L)

    def start_gather(s):
        w, _, cc = units[s]
        idx = plsc.load_gather(ordv, [w + iota])
        src = x.at[:, pl.ds(c0 + cc * CW, CW)]
        pltpu.async_copy(src.at[idx], bufs[s % NBUF], gsems[s % NBUF])

    scats = [None] * NBUF
    for s in range(min(NBUF - 1, len(units))):
        start_gather(s)
    for s in range(len(units)):
        b = s % NBUF
        w, r, cc = units[s]
        src = x.at[:, pl.ds(c0 + cc * CW, CW)]
        idx = plsc.load_gather(ordv, [w + iota])
        pltpu.make_async_copy(src.at[idx], bufs[b], gsems[b]).wait()
        scats[b] = pltpu.async_copy(
            bufs[b].at[pl.ds(0, r)],
            out.at[pl.ds(w, r), pl.ds(c0 + cc * CW, CW)], wsems[b])
        n = s + NBUF - 1
        if n < len(units):
            nb = n % NBUF
            if scats[nb] is not None:
                scats[nb].wait()
                scats[nb] = None
            start_gather(n)
    for b in range(NBUF):
        if scats[b] is not None:
            scats[b].wait()


def kernel(x, ordinals):
    return _sc_gather(x, ordinals)


# speedup vs baseline: 3.9916x; 1.0631x over previous
import functools

import jax
import jax.numpy as jnp
from jax import lax
from jax.experimental import pallas as pl
from jax.experimental.pallas import tpu as pltpu
from jax.experimental.pallas import tpu_sc as plsc

N = 100
D = 131072
NW = 32
WSTRIPE = D // NW
CW = 4096
NCC = WSTRIPE // CW
L = 8
ROW_WINDOWS = tuple((w, 8) for w in range(0, 96, 8)) + ((96, 4),)
NBUF = 3

_mesh = plsc.VectorSubcoreMesh(core_axis_name="c", subcore_axis_name="s")


@functools.partial(
    pl.kernel,
    out_type=jax.ShapeDtypeStruct((N, D), jnp.float32),
    mesh=_mesh,
    compiler_params=pltpu.CompilerParams(needs_layout_passes=False),
    scratch_types=[
        pltpu.VMEM((128,), jnp.int32),
        [pltpu.VMEM((L, CW), jnp.float32) for _ in range(NBUF)],
        [pltpu.SemaphoreType.DMA for _ in range(NBUF)],
        [pltpu.SemaphoreType.DMA for _ in range(NBUF)],
    ],
)
def _sc_gather(x, ords, out, ordv, bufs, gsems, wsems):
    cid = lax.axis_index("c")
    sid = lax.axis_index("s")
    wid = sid * 2 + cid
    c0 = wid * WSTRIPE

    ordv[pl.ds(96, 16)] = jnp.zeros((16,), jnp.int32)
    pltpu.sync_copy(ords, ordv.at[pl.ds(0, N)])

    units = [(w, r, cc) for (w, r) in ROW_WINDOWS for cc in range(NCC)]

    def start_gather(s):
        w, _, cc = units[s]
        src = x.at[:, pl.ds(c0 + cc * CW, CW)]
        pltpu.async_copy(src.at[ordv.at[pl.ds(w, L)]],
                         bufs[s % NBUF], gsems[s % NBUF])

    scats = [None] * NBUF
    for s in range(min(NBUF - 1, len(units))):
        start_gather(s)
    for s in range(len(units)):
        b = s % NBUF
        w, r, cc = units[s]
        src = x.at[:, pl.ds(c0 + cc * CW, CW)]
        pltpu.make_async_copy(src.at[ordv.at[pl.ds(w, L)]],
                              bufs[b], gsems[b]).wait()
        scats[b] = pltpu.async_copy(
            bufs[b].at[pl.ds(0, r)],
            out.at[pl.ds(w, r), pl.ds(c0 + cc * CW, CW)], wsems[b])
        n = s + NBUF - 1
        if n < len(units):
            nb = n % NBUF
            if scats[nb] is not None:
                scats[nb].wait()
                scats[nb] = None
            start_gather(n)
    for b in range(NBUF):
        if scats[b] is not None:
            scats[b].wait()


def kernel(x, ordinals):
    return _sc_gather(x, ordinals)


# speedup vs baseline: 4.1152x; 1.0309x over previous
import functools

import jax
import jax.numpy as jnp
from jax import lax
from jax.experimental import pallas as pl
from jax.experimental.pallas import tpu as pltpu
from jax.experimental.pallas import tpu_sc as plsc

N = 100
D = 131072
NW = 32
WSTRIPE = D // NW
CW = 4096
NCC = WSTRIPE // CW
L = 16
R = 8
UNITS = tuple((w, R) for w in range(0, 96, R)) + ((96, 4),)
NBUF = 3

_mesh = plsc.VectorSubcoreMesh(core_axis_name="c", subcore_axis_name="s")


@functools.partial(
    pl.kernel,
    out_type=jax.ShapeDtypeStruct((N, D), jnp.float32),
    mesh=_mesh,
    compiler_params=pltpu.CompilerParams(needs_layout_passes=False),
    scratch_types=[
        pltpu.VMEM((128,), jnp.int32),
        pltpu.VMEM((12, R), jnp.int32),
        pltpu.VMEM((4,), jnp.int32),
        [pltpu.VMEM((R, CW), jnp.float32) for _ in range(NBUF)],
        pltpu.VMEM((4, CW), jnp.float32),
        [pltpu.SemaphoreType.DMA for _ in range(NBUF)],
        [pltpu.SemaphoreType.DMA for _ in range(NBUF)],
    ],
)
def _sc_gather(x, ords, out, ordv, posv, tailposv, bufs, tailbuf,
               gsems, wsems):
    cid = lax.axis_index("c")
    sid = lax.axis_index("s")
    wid = sid * 2 + cid
    c0 = wid * WSTRIPE

    ordv[pl.ds(96, L)] = jnp.zeros((L,), jnp.int32)
    pltpu.sync_copy(ords, ordv.at[pl.ds(0, N)])

    iota = lax.iota(jnp.int32, L)
    for w in range(0, 112, L):
        iv = w + iota
        valid = iv < N
        ov = plsc.load_gather(ordv, [iv])
        main = jnp.logical_and(valid, ov < 96)
        plsc.store_scatter(posv, [lax.shift_right_logical(ov, 3),
                                  lax.bitwise_and(ov, R - 1)], iv, mask=main)
        tail = jnp.logical_and(valid, ov >= 96)
        plsc.store_scatter(tailposv, [ov - 96], iv, mask=tail)

    out_sl = out.at[:, pl.ds(c0, CW)]

    def _buf(s):
        w, r = UNITS[s]
        return bufs[s % NBUF] if r == R else tailbuf

    def start_read(s):
        w, r = UNITS[s]
        pltpu.async_copy(x.at[pl.ds(w, r), pl.ds(c0, CW)],
                         _buf(s), gsems[s % NBUF])

    scats = [None] * NBUF
    for s in range(min(NBUF - 1, len(UNITS))):
        start_read(s)
    for s in range(len(UNITS)):
        b = s % NBUF
        w, r = UNITS[s]
        pltpu.make_async_copy(x.at[pl.ds(w, r), pl.ds(c0, CW)],
                              _buf(s), gsems[b]).wait()
        if r == R:
            dst = out_sl.at[posv.at[s]]
        else:
            dst = out_sl.at[tailposv]
        scats[b] = pltpu.async_copy(_buf(s), dst, wsems[b])
        n = s + NBUF - 1
        if n < len(UNITS):
            nb = n % NBUF
            if scats[nb] is not None:
                scats[nb].wait()
                scats[nb] = None
            start_read(n)
    for b in range(NBUF):
        if scats[b] is not None:
            scats[b].wait()


def kernel(x, ordinals):
    return _sc_gather(x, ordinals)
